# baseline (speedup 1.0000x reference)
import functools

import jax
import jax.numpy as jnp
from jax import lax
from jax.experimental import pallas as pl
from jax.experimental.pallas import tpu as pltpu
from jax.experimental.pallas import tpu_sc as plsc

B, L = 4096, 200
D = 64
HIDDEN = 256
NUM_CLASSES = 10

NC, NS = 2, 16
NW = NC * NS
BPW = B // NW
IPW = BPW * L
CH0, CH1 = 104, 96
NSLOT = 6

_mesh = plsc.VectorSubcoreMesh(core_axis_name="c", subcore_axis_name="s")


@functools.partial(
    pl.kernel,
    out_type=jax.ShapeDtypeStruct((B, D), jnp.float32),
    mesh=_mesh,
    scratch_types=[
        pltpu.VMEM((IPW,), jnp.int32),
        pltpu.VMEM((NSLOT, L, D), jnp.float32),
        pltpu.VMEM((BPW, D), jnp.float32),
        [pltpu.SemaphoreType.DMA] * NSLOT,
    ],
    compiler_params=pltpu.CompilerParams(
        use_tc_tiling_on_sc=False, needs_layout_passes=False),
)
def _pool(flat_hbm, table_hbm, out_hbm, idx_v, buf_v, out_v, sems):
    wid = lax.axis_index("c") * NS + lax.axis_index("s")
    ibase = wid * IPW

    inv_l = jnp.float32(1.0 / L)

    pltpu.sync_copy(flat_hbm.at[pl.ds(ibase, IPW)], idx_v)

    def issue(r, slot):
        pltpu.async_copy(
            table_hbm.at[idx_v.at[pl.ds(r * L, CH0)]],
            buf_v.at[slot, pl.ds(0, CH0)], sems[slot])
        pltpu.async_copy(
            table_hbm.at[idx_v.at[pl.ds(r * L + CH0, CH1)]],
            buf_v.at[slot, pl.ds(CH0, CH1)], sems[slot])

    def drain(r, slot):
        pltpu.make_async_copy(
            table_hbm.at[idx_v.at[pl.ds(r * L, CH0)]],
            buf_v.at[slot, pl.ds(0, CH0)], sems[slot]).wait()
        pltpu.make_async_copy(
            table_hbm.at[idx_v.at[pl.ds(r * L + CH0, CH1)]],
            buf_v.at[slot, pl.ds(CH0, CH1)], sems[slot]).wait()

    def reduce(r, slot):
        def rbody(j, accs):
            new = list(accs)
            for k in range(4):
                row = 4 * j + k
                for d in range(4):
                    new[d] = new[d] + buf_v[slot, row, pl.ds(16 * d, 16)]
            return tuple(new)

        z = jnp.zeros((16,), jnp.float32)
        accs = lax.fori_loop(0, L // 4, rbody, (z, z, z, z))
        for d in range(4):
            out_v[r, pl.ds(16 * d, 16)] = accs[d] * inv_l

    for s in range(NSLOT):
        issue(s, s)

    n_full = BPW // NSLOT

    def body(g, carry):
        r0 = NSLOT * g
        for s in range(NSLOT):
            drain(r0 + s, s)
            reduce(r0 + s, s)

            @pl.when(r0 + s + NSLOT < BPW)
            def _():
                issue(r0 + s + NSLOT, s)

        return carry

    lax.fori_loop(0, n_full, body, 0)
    for s in range(BPW - n_full * NSLOT):
        drain(n_full * NSLOT + s, s)
        reduce(n_full * NSLOT + s, s)

    pltpu.sync_copy(out_v, out_hbm.at[pl.ds(wid * BPW, BPW)])


NSLOTP = 3


@functools.partial(
    pl.kernel,
    out_type=jax.ShapeDtypeStruct((B, D), jnp.float32),
    mesh=_mesh,
    scratch_types=[
        pltpu.VMEM((IPW + 16,), jnp.int32),
        pltpu.VMEM((NSLOTP, 208), jnp.int32),
        pltpu.VMEM((NSLOTP, L, 2 * D), jnp.float32),
        pltpu.VMEM((BPW, D), jnp.float32),
        [pltpu.SemaphoreType.DMA] * NSLOTP,
    ],
    compiler_params=pltpu.CompilerParams(
        use_tc_tiling_on_sc=False, needs_layout_passes=False),
)
def _pool_pair(flat_hbm, table2_hbm, out_hbm, idx_v, hidx_v, buf_v, out_v, sems):
    wid = lax.axis_index("c") * NS + lax.axis_index("s")
    ibase = wid * IPW

    inv_l = jnp.float32(1.0 / L)

    pltpu.sync_copy(flat_hbm.at[pl.ds(ibase, IPW)], idx_v.at[pl.ds(0, IPW)])

    def issue(r, slot):
        for t in range(13):
            hidx_v[slot, pl.ds(16 * t, 16)] = \
                idx_v[pl.ds(r * L + 16 * t, 16)] >> 1
        pltpu.async_copy(
            table2_hbm.at[hidx_v.at[slot, pl.ds(0, CH0)]],
            buf_v.at[slot, pl.ds(0, CH0)], sems[slot])
        pltpu.async_copy(
            table2_hbm.at[hidx_v.at[slot, pl.ds(CH0, CH1)]],
            buf_v.at[slot, pl.ds(CH0, CH1)], sems[slot])

    def drain(slot):
        pltpu.make_async_copy(
            table2_hbm.at[hidx_v.at[slot, pl.ds(0, CH0)]],
            buf_v.at[slot, pl.ds(0, CH0)], sems[slot]).wait()
        pltpu.make_async_copy(
            table2_hbm.at[hidx_v.at[slot, pl.ds(CH0, CH1)]],
            buf_v.at[slot, pl.ds(CH0, CH1)], sems[slot]).wait()

    def reduce(r, slot):
        def rbody(g, accs):
            a0, a1, a2, a3 = accs
            pv = idx_v[pl.ds(r * L + 8 * g, 16)] & 1
            for k in range(8):
                row = buf_v.at[slot, 8 * g + k]
                take_hi = pv[k] != 0
                s0 = jnp.where(take_hi, row[pl.ds(64, 16)], row[pl.ds(0, 16)])
                s1 = jnp.where(take_hi, row[pl.ds(80, 16)], row[pl.ds(16, 16)])
                s2 = jnp.where(take_hi, row[pl.ds(96, 16)], row[pl.ds(32, 16)])
                s3 = jnp.where(take_hi, row[pl.ds(112, 16)], row[pl.ds(48, 16)])
                a0, a1, a2, a3 = a0 + s0, a1 + s1, a2 + s2, a3 + s3
            return a0, a1, a2, a3

        z = jnp.zeros((16,), jnp.float32)
        accs = lax.fori_loop(0, L // 8, rbody, (z, z, z, z))
        for d in range(4):
            out_v[r, pl.ds(16 * d, 16)] = accs[d] * inv_l

    for s in range(NSLOTP):
        issue(s, s)

    n_full = BPW // NSLOTP

    def body(g, carry):
        r0 = NSLOTP * g
        for s in range(NSLOTP):
            drain(s)
            reduce(r0 + s, s)

            @pl.when(r0 + s + NSLOTP < BPW)
            def _():
                issue(r0 + s + NSLOTP, s)

        return carry

    lax.fori_loop(0, n_full, body, 0)
    for s in range(BPW - n_full * NSLOTP):
        drain(s)
        reduce(n_full * NSLOTP + s, s)

    pltpu.sync_copy(out_v, out_hbm.at[pl.ds(wid * BPW, BPW)])


def _mlp_body(xw_ref, xb_ref, w1a_ref, w1b_ref, b1_ref, w2_ref, b2_ref, o_ref):
    h = jnp.dot(xw_ref[...], w1a_ref[...], preferred_element_type=jnp.float32)
    h = h + jnp.dot(xb_ref[...], w1b_ref[...], preferred_element_type=jnp.float32)
    h = jnp.maximum(h + b1_ref[...], 0.0)
    o = jnp.dot(h, w2_ref[...], preferred_element_type=jnp.float32)
    o_ref[...] = o + b2_ref[...]


_BM = 512


def _mlp(pw, pb, w1a, w1b, b1r, w2p, b2p):
    return pl.pallas_call(
        _mlp_body,
        grid=(B // _BM,),
        in_specs=[
            pl.BlockSpec((_BM, D), lambda i: (i, 0)),
            pl.BlockSpec((_BM, D), lambda i: (i, 0)),
            pl.BlockSpec((D, HIDDEN), lambda i: (0, 0)),
            pl.BlockSpec((D, HIDDEN), lambda i: (0, 0)),
            pl.BlockSpec((1, HIDDEN), lambda i: (0, 0)),
            pl.BlockSpec((HIDDEN, 128), lambda i: (0, 0)),
            pl.BlockSpec((1, 128), lambda i: (0, 0)),
        ],
        out_specs=pl.BlockSpec((_BM, 128), lambda i: (i, 0)),
        out_shape=jax.ShapeDtypeStruct((B, 128), jnp.float32),
    )(pw, pb, w1a, w1b, b1r, w2p, b2p)


def kernel(words, bigram, emb, emb_bigram, W1, b1, W2, b2):
    pooled_w = _pool(words.reshape(-1), emb)
    pooled_b = _pool_pair(bigram.reshape(-1), emb_bigram.reshape(-1, 2 * D))

    w1t = W1.T
    b1r = b1.reshape(1, HIDDEN)
    w2p = jnp.zeros((HIDDEN, 128), W2.dtype).at[:, :NUM_CLASSES].set(W2.T)
    b2p = jnp.zeros((1, 128), b2.dtype).at[0, :NUM_CLASSES].set(b2)
    out = _mlp(pooled_w, pooled_b, w1t[:D], w1t[D:], b1r, w2p, b2p)
    return out[:, :NUM_CLASSES]


# speedup vs baseline: 1.1154x; 1.1154x over previous
import functools

import jax
import jax.numpy as jnp
from jax import lax
from jax.experimental import pallas as pl
from jax.experimental.pallas import tpu as pltpu
from jax.experimental.pallas import tpu_sc as plsc

B, L = 4096, 200
D = 64
HIDDEN = 256
NUM_CLASSES = 10

NC, NS = 2, 16
NW = NC * NS
BPW = B // NW
IPW = BPW * L
CH0, CH1 = 104, 96
NSLOT = 6

_mesh = plsc.VectorSubcoreMesh(core_axis_name="c", subcore_axis_name="s")


@functools.partial(
    pl.kernel,
    out_type=jax.ShapeDtypeStruct((B, D), jnp.float32),
    mesh=_mesh,
    scratch_types=[
        pltpu.VMEM((IPW,), jnp.int32),
        pltpu.VMEM((NSLOT, L, D), jnp.float32),
        pltpu.VMEM((BPW, D), jnp.float32),
        [pltpu.SemaphoreType.DMA] * NSLOT,
    ],
    compiler_params=pltpu.CompilerParams(
        use_tc_tiling_on_sc=False, needs_layout_passes=False),
)
def _pool(flat_hbm, table_hbm, out_hbm, idx_v, buf_v, out_v, sems):
    wid = lax.axis_index("c") * NS + lax.axis_index("s")
    ibase = wid * IPW

    inv_l = jnp.float32(1.0 / L)

    pltpu.sync_copy(flat_hbm.at[pl.ds(ibase, IPW)], idx_v)

    def issue(r, slot):
        pltpu.async_copy(
            table_hbm.at[idx_v.at[pl.ds(r * L, CH0)]],
            buf_v.at[slot, pl.ds(0, CH0)], sems[slot])
        pltpu.async_copy(
            table_hbm.at[idx_v.at[pl.ds(r * L + CH0, CH1)]],
            buf_v.at[slot, pl.ds(CH0, CH1)], sems[slot])

    def drain(r, slot):
        pltpu.make_async_copy(
            table_hbm.at[idx_v.at[pl.ds(r * L, CH0)]],
            buf_v.at[slot, pl.ds(0, CH0)], sems[slot]).wait()
        pltpu.make_async_copy(
            table_hbm.at[idx_v.at[pl.ds(r * L + CH0, CH1)]],
            buf_v.at[slot, pl.ds(CH0, CH1)], sems[slot]).wait()

    def reduce(r, slot):
        def rbody(j, accs):
            new = list(accs)
            for k in range(4):
                row = 4 * j + k
                for d in range(4):
                    new[d] = new[d] + buf_v[slot, row, pl.ds(16 * d, 16)]
            return tuple(new)

        z = jnp.zeros((16,), jnp.float32)
        accs = lax.fori_loop(0, L // 4, rbody, (z, z, z, z))
        for d in range(4):
            out_v[r, pl.ds(16 * d, 16)] = accs[d] * inv_l

    for s in range(NSLOT):
        issue(s, s)

    n_full = BPW // NSLOT

    def body(g, carry):
        r0 = NSLOT * g
        for s in range(NSLOT):
            drain(r0 + s, s)
            reduce(r0 + s, s)

            @pl.when(r0 + s + NSLOT < BPW)
            def _():
                issue(r0 + s + NSLOT, s)

        return carry

    lax.fori_loop(0, n_full, body, 0)
    for s in range(BPW - n_full * NSLOT):
        drain(n_full * NSLOT + s, s)
        reduce(n_full * NSLOT + s, s)

    pltpu.sync_copy(out_v, out_hbm.at[pl.ds(wid * BPW, BPW)])


def _mlp_body(xw_ref, xb_ref, w1a_ref, w1b_ref, b1_ref, w2_ref, b2_ref, o_ref):
    h = jnp.dot(xw_ref[...], w1a_ref[...], preferred_element_type=jnp.float32)
    h = h + jnp.dot(xb_ref[...], w1b_ref[...], preferred_element_type=jnp.float32)
    h = jnp.maximum(h + b1_ref[...], 0.0)
    o = jnp.dot(h, w2_ref[...], preferred_element_type=jnp.float32)
    o_ref[...] = o + b2_ref[...]


_BM = 512


def _mlp(pw, pb, w1a, w1b, b1r, w2p, b2p):
    return pl.pallas_call(
        _mlp_body,
        grid=(B // _BM,),
        in_specs=[
            pl.BlockSpec((_BM, D), lambda i: (i, 0)),
            pl.BlockSpec((_BM, D), lambda i: (i, 0)),
            pl.BlockSpec((D, HIDDEN), lambda i: (0, 0)),
            pl.BlockSpec((D, HIDDEN), lambda i: (0, 0)),
            pl.BlockSpec((1, HIDDEN), lambda i: (0, 0)),
            pl.BlockSpec((HIDDEN, 128), lambda i: (0, 0)),
            pl.BlockSpec((1, 128), lambda i: (0, 0)),
        ],
        out_specs=pl.BlockSpec((_BM, 128), lambda i: (i, 0)),
        out_shape=jax.ShapeDtypeStruct((B, 128), jnp.float32),
    )(pw, pb, w1a, w1b, b1r, w2p, b2p)


def kernel(words, bigram, emb, emb_bigram, W1, b1, W2, b2):
    pooled_w = _pool(words.reshape(-1), emb)
    pooled_b = _pool(bigram.reshape(-1), emb_bigram)

    w1t = W1.T
    b1r = b1.reshape(1, HIDDEN)
    w2p = jnp.zeros((HIDDEN, 128), W2.dtype).at[:, :NUM_CLASSES].set(W2.T)
    b2p = jnp.zeros((1, 128), b2.dtype).at[0, :NUM_CLASSES].set(b2)
    out = _mlp(pooled_w, pooled_b, w1t[:D], w1t[D:], b1r, w2p, b2p)
    return out[:, :NUM_CLASSES]


# speedup vs baseline: 1.1188x; 1.0031x over previous
import functools

import jax
import jax.numpy as jnp
from jax import lax
from jax.experimental import pallas as pl
from jax.experimental.pallas import tpu as pltpu
from jax.experimental.pallas import tpu_sc as plsc

B, L = 4096, 200
D = 64
HIDDEN = 256
NUM_CLASSES = 10

NC, NS = 2, 16
NW = NC * NS
BPW = B // NW
IPW = BPW * L
CH0, CH1 = 104, 96
NSLOT = 6

_mesh = plsc.VectorSubcoreMesh(core_axis_name="c", subcore_axis_name="s")


@functools.partial(
    pl.kernel,
    out_type=jax.ShapeDtypeStruct((B, D), jnp.float32),
    mesh=_mesh,
    scratch_types=[
        pltpu.VMEM((IPW,), jnp.int32),
        pltpu.VMEM((NSLOT, L, D), jnp.float32),
        pltpu.VMEM((BPW, D), jnp.float32),
        [pltpu.SemaphoreType.DMA] * NSLOT,
    ],
    compiler_params=pltpu.CompilerParams(
        use_tc_tiling_on_sc=False, needs_layout_passes=False),
)
def _pool(flat_hbm, table_hbm, out_hbm, idx_v, buf_v, out_v, sems):
    wid = lax.axis_index("c") * NS + lax.axis_index("s")
    ibase = wid * IPW

    inv_l = jnp.float32(1.0 / L)

    pltpu.sync_copy(flat_hbm.at[pl.ds(ibase, IPW)], idx_v)

    def issue(r, slot):
        pltpu.async_copy(
            table_hbm.at[idx_v.at[pl.ds(r * L, CH0)]],
            buf_v.at[slot, pl.ds(0, CH0)], sems[slot])
        pltpu.async_copy(
            table_hbm.at[idx_v.at[pl.ds(r * L + CH0, CH1)]],
            buf_v.at[slot, pl.ds(CH0, CH1)], sems[slot])

    def drain(r, slot):
        pltpu.make_async_copy(
            table_hbm.at[idx_v.at[pl.ds(r * L, CH0)]],
            buf_v.at[slot, pl.ds(0, CH0)], sems[slot]).wait()
        pltpu.make_async_copy(
            table_hbm.at[idx_v.at[pl.ds(r * L + CH0, CH1)]],
            buf_v.at[slot, pl.ds(CH0, CH1)], sems[slot]).wait()

    def reduce(r, slot):
        def rbody(j, accs):
            new = list(accs)
            for k in range(4):
                row = 4 * j + k
                for d in range(4):
                    new[d] = new[d] + buf_v[slot, row, pl.ds(16 * d, 16)]
            return tuple(new)

        z = jnp.zeros((16,), jnp.float32)
        accs = lax.fori_loop(0, L // 4, rbody, (z, z, z, z))
        for d in range(4):
            out_v[r, pl.ds(16 * d, 16)] = accs[d] * inv_l

    for s in range(NSLOT):
        issue(s, s)

    n_full = BPW // NSLOT

    def body(g, carry):
        r0 = NSLOT * g
        for s in range(NSLOT):
            drain(r0 + s, s)
            reduce(r0 + s, s)

            @pl.when(r0 + s + NSLOT < BPW)
            def _():
                issue(r0 + s + NSLOT, s)

        return carry

    lax.fori_loop(0, n_full, body, 0)
    for s in range(BPW - n_full * NSLOT):
        drain(n_full * NSLOT + s, s)
        reduce(n_full * NSLOT + s, s)

    pltpu.sync_copy(out_v, out_hbm.at[pl.ds(wid * BPW, BPW)])


def _mlp_body(xw_ref, xb_ref, w1a_ref, w1b_ref, b1_ref, w2_ref, b2_ref, o_ref):
    h = jnp.dot(xw_ref[...], w1a_ref[...], preferred_element_type=jnp.float32)
    h = h + jnp.dot(xb_ref[...], w1b_ref[...], preferred_element_type=jnp.float32)
    h = jnp.maximum(h + b1_ref[...], 0.0)
    o = jnp.dot(h, w2_ref[...], preferred_element_type=jnp.float32)
    o_ref[...] = o + b2_ref[...]


_BM = 512


def _mlp(pw, pb, w1a, w1b, b1r, w2p, b2p):
    return pl.pallas_call(
        _mlp_body,
        grid=(B // _BM,),
        in_specs=[
            pl.BlockSpec((_BM, D), lambda i: (i, 0)),
            pl.BlockSpec((_BM, D), lambda i: (i, 0)),
            pl.BlockSpec((D, HIDDEN), lambda i: (0, 0)),
            pl.BlockSpec((D, HIDDEN), lambda i: (0, 0)),
            pl.BlockSpec((1, HIDDEN), lambda i: (0, 0)),
            pl.BlockSpec((HIDDEN, 128), lambda i: (0, 0)),
            pl.BlockSpec((1, 128), lambda i: (0, 0)),
        ],
        out_specs=pl.BlockSpec((_BM, 128), lambda i: (i, 0)),
        out_shape=jax.ShapeDtypeStruct((B, 128), jnp.float32),
    )(pw, pb, w1a, w1b, b1r, w2p, b2p)


def kernel(words, bigram, emb, emb_bigram, W1, b1, W2, b2):
    pooled_b = _pool(bigram.reshape(-1), emb_bigram)
    pooled_w = _pool(words.reshape(-1), emb)

    w1t = W1.T
    b1r = b1.reshape(1, HIDDEN)
    w2p = jnp.zeros((HIDDEN, 128), W2.dtype).at[:, :NUM_CLASSES].set(W2.T)
    b2p = jnp.zeros((1, 128), b2.dtype).at[0, :NUM_CLASSES].set(b2)
    out = _mlp(pooled_w, pooled_b, w1t[:D], w1t[D:], b1r, w2p, b2p)
    return out[:, :NUM_CLASSES]
